# baseline (speedup 1.0000x reference)
import functools

import jax
import jax.numpy as jnp
from jax import lax
from jax.experimental import pallas as pl
from jax.experimental.pallas import tpu as pltpu

TOP_K = 8
NUM_EXPERTS = 64


def _router_kernel(x_ref, w_ref, logits_ref, topv_ref, topi_ref):
    x = x_ref[...]
    w = w_ref[...]
    lt = lax.dot_general(
        w, x,
        dimension_numbers=(((1,), (1,)), ((), ())),
        preferred_element_type=jnp.float32,
    )
    logits_ref[...] = lt.T

    t = lt.shape[1]
    e = jnp.exp(lt - jnp.max(lt, axis=0, keepdims=True))
    scores = e / jnp.sum(e, axis=0, keepdims=True)
    iota = lax.broadcasted_iota(jnp.int32, (NUM_EXPERTS, t), 0)
    work = scores
    vals = []
    idxs = []
    for _ in range(TOP_K):
        m = jnp.max(work, axis=0, keepdims=True)
        idx = jnp.min(jnp.where(work == m, iota, NUM_EXPERTS), axis=0,
                      keepdims=True)
        vals.append(m)
        idxs.append(idx)
        work = jnp.where(iota == idx, -1.0, work)
    topvals = jnp.concatenate(vals, axis=0)
    topidx = jnp.concatenate(idxs, axis=0)
    topv = topvals / jnp.sum(topvals, axis=0, keepdims=True)
    topv_ref[...] = topv.T
    topi_ref[...] = topidx.T


@functools.partial(jax.jit, static_argnames=("block_t",))
def _router(x_flat, weight, block_t=512):
    n_tokens, hidden = x_flat.shape
    grid = (n_tokens // block_t,)
    return pl.pallas_call(
        _router_kernel,
        grid=grid,
        in_specs=[
            pl.BlockSpec((block_t, hidden), lambda i: (i, 0)),
            pl.BlockSpec((NUM_EXPERTS, hidden), lambda i: (0, 0)),
        ],
        out_specs=[
            pl.BlockSpec((block_t, NUM_EXPERTS), lambda i: (i, 0)),
            pl.BlockSpec((block_t, TOP_K), lambda i: (i, 0)),
            pl.BlockSpec((block_t, TOP_K), lambda i: (i, 0)),
        ],
        out_shape=[
            jax.ShapeDtypeStruct((n_tokens, NUM_EXPERTS), jnp.float32),
            jax.ShapeDtypeStruct((n_tokens, TOP_K), jnp.float32),
            jax.ShapeDtypeStruct((n_tokens, TOP_K), jnp.int32),
        ],
    )(x_flat, weight)


def kernel(x, weight):
    hidden = weight.shape[1]
    x_flat = x.reshape(-1, hidden)
    logits, topv, topi = _router(x_flat, weight)
    return (logits, topv, topi)


# speedup vs baseline: 1.1542x; 1.1542x over previous
import functools

import jax
import jax.numpy as jnp
from jax import lax
from jax.experimental import pallas as pl
from jax.experimental.pallas import tpu as pltpu

TOP_K = 8
NUM_EXPERTS = 64


def _router_kernel(x_ref, w_ref, logits_ref, topv_ref, topi_ref):
    x = x_ref[...]
    w = w_ref[...]
    lt = lax.dot_general(
        w, x,
        dimension_numbers=(((1,), (1,)), ((), ())),
        preferred_element_type=jnp.float32,
    )
    logits_ref[...] = lt.T

    t = lt.shape[1]
    e = jnp.exp(lt - jnp.max(lt, axis=0, keepdims=True))
    scores = e / jnp.sum(e, axis=0, keepdims=True)
    iota = lax.broadcasted_iota(jnp.int32, (NUM_EXPERTS, t), 0)
    work = scores
    vals = []
    idxs = []
    for _ in range(TOP_K):
        m = jnp.max(work, axis=0, keepdims=True)
        idx = jnp.min(jnp.where(work == m, iota, NUM_EXPERTS), axis=0,
                      keepdims=True)
        vals.append(m)
        idxs.append(idx)
        work = jnp.where(iota == idx, -1.0, work)
    topvals = jnp.concatenate(vals, axis=0)
    topidx = jnp.concatenate(idxs, axis=0)
    topv = topvals / jnp.sum(topvals, axis=0, keepdims=True)
    topv_ref[...] = topv.T
    topi_ref[...] = topidx.T


@functools.partial(jax.jit, static_argnames=("block_t",))
def _router(x_flat, weight, block_t=1024):
    n_tokens, hidden = x_flat.shape
    grid = (n_tokens // block_t,)
    return pl.pallas_call(
        _router_kernel,
        grid=grid,
        in_specs=[
            pl.BlockSpec((block_t, hidden), lambda i: (i, 0)),
            pl.BlockSpec((NUM_EXPERTS, hidden), lambda i: (0, 0)),
        ],
        out_specs=[
            pl.BlockSpec((block_t, NUM_EXPERTS), lambda i: (i, 0)),
            pl.BlockSpec((block_t, TOP_K), lambda i: (i, 0)),
            pl.BlockSpec((block_t, TOP_K), lambda i: (i, 0)),
        ],
        out_shape=[
            jax.ShapeDtypeStruct((n_tokens, NUM_EXPERTS), jnp.float32),
            jax.ShapeDtypeStruct((n_tokens, TOP_K), jnp.float32),
            jax.ShapeDtypeStruct((n_tokens, TOP_K), jnp.int32),
        ],
    )(x_flat, weight)


def kernel(x, weight):
    hidden = weight.shape[1]
    x_flat = x.reshape(-1, hidden)
    logits, topv, topi = _router(x_flat, weight)
    return (logits, topv, topi)


# speedup vs baseline: 1.2019x; 1.0413x over previous
import functools

import jax
import jax.numpy as jnp
from jax import lax
from jax.experimental import pallas as pl
from jax.experimental.pallas import tpu as pltpu

TOP_K = 8
NUM_EXPERTS = 64


def _router_kernel(x_ref, w_ref, logits_ref, topv_ref, topi_ref):
    x = x_ref[...]
    w = w_ref[...]
    lt = lax.dot_general(
        w, x,
        dimension_numbers=(((1,), (1,)), ((), ())),
        preferred_element_type=jnp.float32,
    )
    logits_ref[...] = lt.T

    t = lt.shape[1]
    e = jnp.exp(lt - jnp.max(lt, axis=0, keepdims=True))
    scores = e / jnp.sum(e, axis=0, keepdims=True)
    iota = lax.broadcasted_iota(jnp.int32, (NUM_EXPERTS, t), 0)
    work = scores
    vals = []
    idxs = []
    for _ in range(TOP_K):
        m = jnp.max(work, axis=0, keepdims=True)
        idx = jnp.min(jnp.where(work == m, iota, NUM_EXPERTS), axis=0,
                      keepdims=True)
        vals.append(m)
        idxs.append(idx)
        work = jnp.where(iota == idx, -1.0, work)
    topvals = jnp.concatenate(vals, axis=0)
    topidx = jnp.concatenate(idxs, axis=0)
    topv = topvals / jnp.sum(topvals, axis=0, keepdims=True)
    topv_ref[...] = topv.T
    topi_ref[...] = topidx.T


@functools.partial(jax.jit, static_argnames=("block_t",))
def _router(x_flat, weight, block_t=2048):
    n_tokens, hidden = x_flat.shape
    grid = (n_tokens // block_t,)
    return pl.pallas_call(
        _router_kernel,
        grid=grid,
        in_specs=[
            pl.BlockSpec((block_t, hidden), lambda i: (i, 0)),
            pl.BlockSpec((NUM_EXPERTS, hidden), lambda i: (0, 0)),
        ],
        out_specs=[
            pl.BlockSpec((block_t, NUM_EXPERTS), lambda i: (i, 0)),
            pl.BlockSpec((block_t, TOP_K), lambda i: (i, 0)),
            pl.BlockSpec((block_t, TOP_K), lambda i: (i, 0)),
        ],
        out_shape=[
            jax.ShapeDtypeStruct((n_tokens, NUM_EXPERTS), jnp.float32),
            jax.ShapeDtypeStruct((n_tokens, TOP_K), jnp.float32),
            jax.ShapeDtypeStruct((n_tokens, TOP_K), jnp.int32),
        ],
    )(x_flat, weight)


def kernel(x, weight):
    hidden = weight.shape[1]
    x_flat = x.reshape(-1, hidden)
    logits, topv, topi = _router(x_flat, weight)
    return (logits, topv, topi)
